# baseline (speedup 1.0000x reference)
import functools

import jax
import jax.numpy as jnp
from jax import lax
from jax.experimental import pallas as pl
from jax.experimental.pallas import tpu as pltpu
from jax.experimental.pallas import tpu_sc as plsc

B, T = 4096, 200
V, E, H = 1000000, 64, 32

NC, NS = 2, 16
NW = NC * NS
BSTRIPE = B // NW
CHUNK = BSTRIPE
NBUF = 4
LANES = 16

BT_TILE = 4096
NB = B // BT_TILE


def _gather_body(table_hbm, x_hbm, out_hbm, idx_v, idxc_v, rows_v, *sems):
    wid = lax.axis_index("s") * NC + lax.axis_index("c")
    stripe = wid * BSTRIPE

    pltpu.sync_copy(x_hbm.at[pl.ds(stripe * T, BSTRIPE * T)], idx_v)

    def build_col(t, slot):
        for j in range(BSTRIPE // LANES):
            pos = (lax.iota(jnp.int32, LANES) + LANES * j) * T + t
            idxc_v[slot, pl.ds(LANES * j, LANES)] = plsc.load_gather(
                idx_v, [pos])

    def gather_copy(slot):
        return pltpu.make_async_copy(
            table_hbm.at[idxc_v.at[slot]], rows_v.at[slot], sems[slot])

    for b in range(NBUF):
        build_col(b, b)
        gather_copy(b).start()

    def body(i, carry):
        for b in range(NBUF):
            t = i * NBUF + b
            gather_copy(b).wait()
            pltpu.sync_copy(
                rows_v.at[b], out_hbm.at[pl.ds(t * B + stripe, CHUNK)])
            nxt = t + NBUF

            @pl.when(nxt < T)
            def _():
                build_col(nxt, b)
                gather_copy(b).start()
        return carry

    lax.fori_loop(0, T // NBUF, body, 0)


@functools.cache
def _make_sc_gather():
    return pl.kernel(
        _gather_body,
        out_type=jax.ShapeDtypeStruct((B * T, E), jnp.float32),
        mesh=plsc.VectorSubcoreMesh(core_axis_name="c", subcore_axis_name="s"),
        scratch_types=[
            pltpu.VMEM((BSTRIPE * T,), jnp.int32),
            pltpu.VMEM((NBUF, CHUNK), jnp.int32),
            pltpu.VMEM((NBUF, CHUNK, E), jnp.float32),
        ] + [pltpu.SemaphoreType.DMA] * NBUF,
        compiler_params=pltpu.CompilerParams(
            use_tc_tiling_on_sc=False, needs_layout_passes=False),
    )


B2 = B // 2
E2, H2 = 2 * E, 2 * H


def _lstm_body(e0_ref, e1_ref, e2_ref, e3_ref, wx_ref, wh_ref, b_ref,
               out_ref, h_scr, c_scr):
    t = pl.program_id(0)

    @pl.when(t == 0)
    def _():
        h_scr[...] = jnp.zeros_like(h_scr)
        c_scr[...] = jnp.zeros_like(c_scr)

    xt = jnp.concatenate(
        [e0_ref[0, 0], e1_ref[0, 0], e2_ref[0, 0], e3_ref[0, 0]],
        axis=0)
    h = h_scr[...]

    def gate(k):
        return (
            lax.dot_general(xt, wx_ref[k], (((1,), (0,)), ((), ())),
                            preferred_element_type=jnp.float32)
            + lax.dot_general(h, wh_ref[k], (((1,), (0,)), ((), ())),
                              preferred_element_type=jnp.float32)
            + b_ref[k]
        )

    i = jax.nn.sigmoid(gate(0))
    f = jax.nn.sigmoid(gate(1))
    g = jnp.tanh(gate(2))
    o = jax.nn.sigmoid(gate(3))
    c_new = f * c_scr[...] + i * g
    h_new = o * jnp.tanh(c_new)
    c_scr[...] = c_new
    h_scr[...] = h_new

    @pl.when(t == T - 1)
    def _():
        out_ref[...] = h_new


NSTREAM = 4
BQ = B2 // NSTREAM


def _lstm(emb_q, wx2, wh2, bias2, interpret=False):
    qspec = [
        pl.BlockSpec((1, 1, BQ, E2), lambda t, j=j: (t, j, 0, 0))
        for j in range(NSTREAM)
    ]
    return pl.pallas_call(
        _lstm_body,
        grid=(T,),
        in_specs=qspec + [
            pl.BlockSpec((4, E2, H2), lambda t: (0, 0, 0)),
            pl.BlockSpec((4, H2, H2), lambda t: (0, 0, 0)),
            pl.BlockSpec((4, 1, H2), lambda t: (0, 0, 0)),
        ],
        out_specs=pl.BlockSpec((B2, H2), lambda t: (0, 0)),
        out_shape=jax.ShapeDtypeStruct((B2, H2), jnp.float32),
        scratch_shapes=[
            pltpu.VMEM((B2, H2), jnp.float32),
            pltpu.VMEM((B2, H2), jnp.float32),
        ],
        compiler_params=pltpu.CompilerParams(
            dimension_semantics=("arbitrary",)),
        interpret=interpret,
    )(emb_q, emb_q, emb_q, emb_q, wx2, wh2, bias2)


def _blockdiag(w):
    k4, K, Hh = w.shape
    z = jnp.zeros((k4, K, Hh), w.dtype)
    top = jnp.concatenate([w, z], axis=2)
    bot = jnp.concatenate([z, w], axis=2)
    return jnp.concatenate([top, bot], axis=1)


def kernel(x, emb, W_ih, W_hh, b_ih, b_hh):
    emb_q = _make_sc_gather()(
        emb, x.astype(jnp.int32).reshape(B * T)).reshape(T, NSTREAM, BQ, E2)
    wx = jnp.transpose(W_ih.reshape(4, H, E), (0, 2, 1))
    wh = jnp.transpose(W_hh.reshape(4, H, H), (0, 2, 1))
    bias = (b_ih + b_hh).reshape(4, 1, H)
    bias2 = jnp.concatenate([bias, bias], axis=2)
    out = _lstm(emb_q, _blockdiag(wx), _blockdiag(wh), bias2)
    return out.reshape(B, H)


# speedup vs baseline: 1.0439x; 1.0439x over previous
import functools

import jax
import jax.numpy as jnp
from jax import lax
from jax.experimental import pallas as pl
from jax.experimental.pallas import tpu as pltpu
from jax.experimental.pallas import tpu_sc as plsc

B, T = 4096, 200
V, E, H = 1000000, 64, 32

NC, NS = 2, 16
NW = NC * NS
TCH = 20
NCHT = T // TCH
CROWS = TCH * B
RPW = CROWS // NW
CHUNK = 128
SUB = RPW // CHUNK
NBUF = 4

B2 = B // 2
E2, H2 = 2 * E, 2 * H
NSTREAM = 4
BQ = B2 // NSTREAM


def _gather_body(table_hbm, idx_hbm, out_hbm, idx_v, rows_v, *sems):
    wid = lax.axis_index("s") * NC + lax.axis_index("c")
    base = wid * RPW

    pltpu.sync_copy(idx_hbm.at[pl.ds(base, RPW)], idx_v)

    def gather_copy(c, slot):
        return pltpu.make_async_copy(
            table_hbm.at[idx_v.at[pl.ds(c * CHUNK, CHUNK)]],
            rows_v.at[slot], sems[slot])

    for b in range(NBUF):
        gather_copy(b, b).start()

    def body(i, carry):
        for b in range(NBUF):
            c = i * NBUF + b
            gather_copy(c, b).wait()
            pltpu.sync_copy(
                rows_v.at[b], out_hbm.at[pl.ds(base + c * CHUNK, CHUNK)])
            nxt = c + NBUF

            @pl.when(nxt < SUB)
            def _():
                gather_copy(nxt, b).start()
        return carry

    lax.fori_loop(0, SUB // NBUF, body, 0)


@functools.cache
def _make_sc_gather():
    return pl.kernel(
        _gather_body,
        out_type=jax.ShapeDtypeStruct((CROWS, E), jnp.float32),
        mesh=plsc.VectorSubcoreMesh(core_axis_name="c", subcore_axis_name="s"),
        scratch_types=[
            pltpu.VMEM((RPW,), jnp.int32),
            pltpu.VMEM((NBUF, CHUNK, E), jnp.float32),
        ] + [pltpu.SemaphoreType.DMA] * NBUF,
        compiler_params=pltpu.CompilerParams(
            use_tc_tiling_on_sc=False, needs_layout_passes=False),
    )


def _lstm_body(e0_ref, e1_ref, e2_ref, e3_ref, wx_ref, wh_ref, b_ref,
               hin_ref, cin_ref, hout_ref, cout_ref, h_scr, c_scr):
    t = pl.program_id(0)

    @pl.when(t == 0)
    def _():
        h_scr[...] = hin_ref[...]
        c_scr[...] = cin_ref[...]

    xt = jnp.concatenate(
        [e0_ref[0, 0], e1_ref[0, 0], e2_ref[0, 0], e3_ref[0, 0]],
        axis=0)
    h = h_scr[...]

    def gate(k):
        return (
            lax.dot_general(xt, wx_ref[k], (((1,), (0,)), ((), ())),
                            preferred_element_type=jnp.float32)
            + lax.dot_general(h, wh_ref[k], (((1,), (0,)), ((), ())),
                              preferred_element_type=jnp.float32)
            + b_ref[k]
        )

    i = jax.nn.sigmoid(gate(0))
    f = jax.nn.sigmoid(gate(1))
    g = jnp.tanh(gate(2))
    o = jax.nn.sigmoid(gate(3))
    c_new = f * c_scr[...] + i * g
    h_new = o * jnp.tanh(c_new)
    c_scr[...] = c_new
    h_scr[...] = h_new

    @pl.when(t == TCH - 1)
    def _():
        hout_ref[...] = h_new
        cout_ref[...] = c_new


def _lstm_chunk(emb_q, wx2, wh2, bias2, h, c, interpret=False):
    qspec = [
        pl.BlockSpec((1, 1, BQ, E2), lambda t, j=j: (t, j, 0, 0))
        for j in range(NSTREAM)
    ]
    state = pl.BlockSpec((B2, H2), lambda t: (0, 0))
    return pl.pallas_call(
        _lstm_body,
        grid=(TCH,),
        in_specs=qspec + [
            pl.BlockSpec((4, E2, H2), lambda t: (0, 0, 0)),
            pl.BlockSpec((4, H2, H2), lambda t: (0, 0, 0)),
            pl.BlockSpec((4, 1, H2), lambda t: (0, 0, 0)),
            state, state,
        ],
        out_specs=(state, state),
        out_shape=(jax.ShapeDtypeStruct((B2, H2), jnp.float32),
                   jax.ShapeDtypeStruct((B2, H2), jnp.float32)),
        scratch_shapes=[
            pltpu.VMEM((B2, H2), jnp.float32),
            pltpu.VMEM((B2, H2), jnp.float32),
        ],
        compiler_params=pltpu.CompilerParams(
            dimension_semantics=("arbitrary",)),
        interpret=interpret,
    )(emb_q, emb_q, emb_q, emb_q, wx2, wh2, bias2, h, c)


def _blockdiag(w):
    k4, K, Hh = w.shape
    z = jnp.zeros((k4, K, Hh), w.dtype)
    top = jnp.concatenate([w, z], axis=2)
    bot = jnp.concatenate([z, w], axis=2)
    return jnp.concatenate([top, bot], axis=1)


def kernel(x, emb, W_ih, W_hh, b_ih, b_hh):
    x_tm = jnp.transpose(x).astype(jnp.int32).reshape(T * B)
    gather = _make_sc_gather()
    wx = jnp.transpose(W_ih.reshape(4, H, E), (0, 2, 1))
    wh = jnp.transpose(W_hh.reshape(4, H, H), (0, 2, 1))
    bias = (b_ih + b_hh).reshape(4, 1, H)
    wx2, wh2 = _blockdiag(wx), _blockdiag(wh)
    bias2 = jnp.concatenate([bias, bias], axis=2)

    h = jnp.zeros((B2, H2), jnp.float32)
    c = jnp.zeros((B2, H2), jnp.float32)
    for k in range(NCHT):
        idx_k = lax.slice_in_dim(x_tm, k * CROWS, (k + 1) * CROWS)
        emb_q = gather(emb, idx_k).reshape(TCH, NSTREAM, BQ, E2)
        h, c = _lstm_chunk(emb_q, wx2, wh2, bias2, h, c)
    return h.reshape(B, H)


# speedup vs baseline: 1.0935x; 1.0475x over previous
import functools

import jax
import jax.numpy as jnp
from jax import lax
from jax.experimental import pallas as pl
from jax.experimental.pallas import tpu as pltpu
from jax.experimental.pallas import tpu_sc as plsc

B, T = 4096, 200
V, E, H = 1000000, 64, 32

NC, NS = 2, 16
NW = NC * NS
TCH = 40
NCHT = T // TCH
TT = TCH // 8
CROWS = TCH * B
CHUNK = 128
NBUF = 4

B2 = B // 2
E2, H2 = 2 * E, 2 * H
TPB = 2


def _gather_body(tc0, table_hbm, x4_hbm, out_hbm, idx_v, rows_v, *sems):
    wid = lax.axis_index("s") * NC + lax.axis_index("c")

    pltpu.sync_copy(x4_hbm.at[pl.ds(tc0, TT), wid], idx_v)

    def gather_copy(c, slot):
        return pltpu.make_async_copy(
            table_hbm.at[idx_v.at[c // 8, c % 8]],
            rows_v.at[slot], sems[slot])

    for b in range(NBUF):
        gather_copy(b, b).start()

    def body(i, carry):
        for b in range(NBUF):
            c = i * NBUF + b
            gather_copy(c, b).wait()
            pltpu.sync_copy(
                rows_v.at[b],
                out_hbm.at[pl.ds(c * B + wid * CHUNK, CHUNK)])
            nxt = c + NBUF

            @pl.when(nxt < TCH)
            def _():
                gather_copy(nxt, b).start()
        return carry

    lax.fori_loop(0, TCH // NBUF, body, 0)


@functools.cache
def _make_sc_gather(k):
    return pl.kernel(
        functools.partial(_gather_body, k * TT),
        out_type=jax.ShapeDtypeStruct((CROWS, E), jnp.float32),
        mesh=plsc.VectorSubcoreMesh(core_axis_name="c", subcore_axis_name="s"),
        scratch_types=[
            pltpu.VMEM((TT, 8, CHUNK), jnp.int32),
            pltpu.VMEM((NBUF, CHUNK, E), jnp.float32),
        ] + [pltpu.SemaphoreType.DMA] * NBUF,
        compiler_params=pltpu.CompilerParams(
            use_tc_tiling_on_sc=False, needs_layout_passes=False),
    )


def _lstm_body(emb_ref, wx_ref, wh_ref, b_ref,
               hin_ref, cin_ref, hout_ref, cout_ref, h_scr, c_scr):
    tt = pl.program_id(0)

    @pl.when(tt == 0)
    def _():
        h_scr[...] = hin_ref[...]
        c_scr[...] = cin_ref[...]

    h = h_scr[...]
    c = c_scr[...]
    for p in range(TPB):
        xt = emb_ref[p]

        def gate(k, h=h, xt=xt):
            return (
                lax.dot_general(xt, wx_ref[k], (((1,), (0,)), ((), ())),
                                preferred_element_type=jnp.float32)
                + lax.dot_general(h, wh_ref[k], (((1,), (0,)), ((), ())),
                                  preferred_element_type=jnp.float32)
                + b_ref[k]
            )

        i = jax.nn.sigmoid(gate(0))
        f = jax.nn.sigmoid(gate(1))
        g = jnp.tanh(gate(2))
        o = jax.nn.sigmoid(gate(3))
        c = f * c + i * g
        h = o * jnp.tanh(c)
    h_scr[...] = h
    c_scr[...] = c

    @pl.when(tt == TCH // TPB - 1)
    def _():
        hout_ref[...] = h
        cout_ref[...] = c


def _lstm_chunk(emb_c, wx2, wh2, bias2, h, c, interpret=False):
    state = pl.BlockSpec((B2, H2), lambda t: (0, 0))
    return pl.pallas_call(
        _lstm_body,
        grid=(TCH // TPB,),
        in_specs=[
            pl.BlockSpec((TPB, B2, E2), lambda t: (t, 0, 0)),
            pl.BlockSpec((4, E2, H2), lambda t: (0, 0, 0)),
            pl.BlockSpec((4, H2, H2), lambda t: (0, 0, 0)),
            pl.BlockSpec((4, 1, H2), lambda t: (0, 0, 0)),
            state, state,
        ],
        out_specs=(state, state),
        out_shape=(jax.ShapeDtypeStruct((B2, H2), jnp.float32),
                   jax.ShapeDtypeStruct((B2, H2), jnp.float32)),
        scratch_shapes=[
            pltpu.VMEM((B2, H2), jnp.float32),
            pltpu.VMEM((B2, H2), jnp.float32),
        ],
        compiler_params=pltpu.CompilerParams(
            dimension_semantics=("arbitrary",)),
        interpret=interpret,
    )(emb_c, wx2, wh2, bias2, h, c)


def _blockdiag(w):
    k4, K, Hh = w.shape
    z = jnp.zeros((k4, K, Hh), w.dtype)
    top = jnp.concatenate([w, z], axis=2)
    bot = jnp.concatenate([z, w], axis=2)
    return jnp.concatenate([top, bot], axis=1)


def kernel(x, emb, W_ih, W_hh, b_ih, b_hh):
    x4 = jnp.transpose(
        x.astype(jnp.int32).reshape(NW, CHUNK, T // 8, 8), (2, 0, 3, 1))
    wx = jnp.transpose(W_ih.reshape(4, H, E), (0, 2, 1))
    wh = jnp.transpose(W_hh.reshape(4, H, H), (0, 2, 1))
    bias = (b_ih + b_hh).reshape(4, 1, H)
    wx2, wh2 = _blockdiag(wx), _blockdiag(wh)
    bias2 = jnp.concatenate([bias, bias], axis=2)

    h = jnp.zeros((B2, H2), jnp.float32)
    c = jnp.zeros((B2, H2), jnp.float32)
    for k in range(NCHT):
        emb_c = _make_sc_gather(k)(emb, x4).reshape(TCH, B2, E2)
        h, c = _lstm_chunk(emb_c, wx2, wh2, bias2, h, c)
    return h.reshape(B, H)


# speedup vs baseline: 1.1489x; 1.0507x over previous
import functools

import jax
import jax.numpy as jnp
from jax import lax
from jax.experimental import pallas as pl
from jax.experimental.pallas import tpu as pltpu
from jax.experimental.pallas import tpu_sc as plsc

B, T = 4096, 200
V, E, H = 1000000, 64, 32

NC, NS = 2, 16
NW = NC * NS
TCH = 40
NCHT = T // TCH
TT = TCH // 8
CROWS = TCH * B
CHUNK = 128
NBUF = 4

B2 = B // 2
E2, H2 = 2 * E, 2 * H
TPB = 2


def _gather_body(tc0, table_hbm, x4_hbm, out_hbm, idx_v, rows_v, *sems):
    wid = lax.axis_index("s") * NC + lax.axis_index("c")

    pltpu.sync_copy(x4_hbm.at[pl.ds(tc0, TT), wid], idx_v)

    def gather_copy(c, slot):
        return pltpu.make_async_copy(
            table_hbm.at[idx_v.at[c // 8, c % 8]],
            rows_v.at[slot], sems[slot])

    for b in range(NBUF):
        gather_copy(b, b).start()

    def body(i, carry):
        for b in range(NBUF):
            c = i * NBUF + b
            gather_copy(c, b).wait()
            pltpu.sync_copy(
                rows_v.at[b, :, pl.ds(0, E)],
                out_hbm.at[pl.ds(c * B + wid * CHUNK, CHUNK)])
            nxt = c + NBUF

            @pl.when(nxt < TCH)
            def _():
                gather_copy(nxt, b).start()
        return carry

    lax.fori_loop(0, TCH // NBUF, body, 0)


@functools.cache
def _make_sc_gather(k):
    return pl.kernel(
        functools.partial(_gather_body, k * TT),
        out_type=jax.ShapeDtypeStruct((CROWS, E), jnp.float32),
        mesh=plsc.VectorSubcoreMesh(core_axis_name="c", subcore_axis_name="s"),
        scratch_types=[
            pltpu.VMEM((TT, 8, CHUNK), jnp.int32),
            pltpu.VMEM((NBUF, CHUNK, 2 * E), jnp.float32),
        ] + [pltpu.SemaphoreType.DMA] * NBUF,
        compiler_params=pltpu.CompilerParams(
            use_tc_tiling_on_sc=False, needs_layout_passes=False),
    )


def _lstm_body(emb_ref, wx_ref, wh_ref, b_ref,
               hin_ref, cin_ref, hout_ref, cout_ref, h_scr, c_scr):
    tt = pl.program_id(0)

    @pl.when(tt == 0)
    def _():
        h_scr[...] = hin_ref[...]
        c_scr[...] = cin_ref[...]

    h = h_scr[...]
    c = c_scr[...]
    for p in range(TPB):
        xt = emb_ref[p]

        def gate(k, h=h, xt=xt):
            return (
                lax.dot_general(xt, wx_ref[k], (((1,), (0,)), ((), ())),
                                preferred_element_type=jnp.float32)
                + lax.dot_general(h, wh_ref[k], (((1,), (0,)), ((), ())),
                                  preferred_element_type=jnp.float32)
                + b_ref[k]
            )

        i = jax.nn.sigmoid(gate(0))
        f = jax.nn.sigmoid(gate(1))
        g = jnp.tanh(gate(2))
        o = jax.nn.sigmoid(gate(3))
        c = f * c + i * g
        h = o * jnp.tanh(c)
    h_scr[...] = h
    c_scr[...] = c

    @pl.when(tt == TCH // TPB - 1)
    def _():
        hout_ref[...] = h
        cout_ref[...] = c


def _lstm_chunk(emb_c, wx2, wh2, bias2, h, c, interpret=False):
    state = pl.BlockSpec((B2, H2), lambda t: (0, 0))
    return pl.pallas_call(
        _lstm_body,
        grid=(TCH // TPB,),
        in_specs=[
            pl.BlockSpec((TPB, B2, E2), lambda t: (t, 0, 0)),
            pl.BlockSpec((4, E2, H2), lambda t: (0, 0, 0)),
            pl.BlockSpec((4, H2, H2), lambda t: (0, 0, 0)),
            pl.BlockSpec((4, 1, H2), lambda t: (0, 0, 0)),
            state, state,
        ],
        out_specs=(state, state),
        out_shape=(jax.ShapeDtypeStruct((B2, H2), jnp.float32),
                   jax.ShapeDtypeStruct((B2, H2), jnp.float32)),
        scratch_shapes=[
            pltpu.VMEM((B2, H2), jnp.float32),
            pltpu.VMEM((B2, H2), jnp.float32),
        ],
        compiler_params=pltpu.CompilerParams(
            dimension_semantics=("arbitrary",)),
        interpret=interpret,
    )(emb_c, wx2, wh2, bias2, h, c)


def _blockdiag(w):
    k4, K, Hh = w.shape
    z = jnp.zeros((k4, K, Hh), w.dtype)
    top = jnp.concatenate([w, z], axis=2)
    bot = jnp.concatenate([z, w], axis=2)
    return jnp.concatenate([top, bot], axis=1)


def kernel(x, emb, W_ih, W_hh, b_ih, b_hh):
    x4 = jnp.transpose(
        x.astype(jnp.int32).reshape(NW, CHUNK, T // 8, 8), (2, 0, 3, 1))
    wx = jnp.transpose(W_ih.reshape(4, H, E), (0, 2, 1))
    wh = jnp.transpose(W_hh.reshape(4, H, H), (0, 2, 1))
    bias = (b_ih + b_hh).reshape(4, 1, H)
    wx2, wh2 = _blockdiag(wx), _blockdiag(wh)
    bias2 = jnp.concatenate([bias, bias], axis=2)

    emb128 = jnp.concatenate([emb, jnp.zeros((V, E), jnp.float32)], axis=1)

    h = jnp.zeros((B2, H2), jnp.float32)
    c = jnp.zeros((B2, H2), jnp.float32)
    for k in range(NCHT):
        emb_c = _make_sc_gather(k)(emb128, x4).reshape(TCH, B2, E2)
        h, c = _lstm_chunk(emb_c, wx2, wh2, bias2, h, c)
    return h.reshape(B, H)


# speedup vs baseline: 1.1979x; 1.0426x over previous
import functools

import jax
import jax.numpy as jnp
from jax import lax
from jax.experimental import pallas as pl
from jax.experimental.pallas import tpu as pltpu
from jax.experimental.pallas import tpu_sc as plsc

B, T = 4096, 200
V, E, H = 1000000, 64, 32

NC, NS = 2, 16
NW = NC * NS
TCH = 40
NCHT = T // TCH
TT = TCH // 8
CROWS = TCH * B
CHUNK = 128
NBUF = 5

B2 = B // 2
E2, H2 = 2 * E, 2 * H
TPB = 4


def _gather_body(tc0, table_hbm, x4_hbm, out_hbm, idx_v, rows_v, *sems):
    wid = lax.axis_index("s") * NC + lax.axis_index("c")

    pltpu.sync_copy(x4_hbm.at[pl.ds(tc0, TT), wid], idx_v)

    def gather_copy(c, slot):
        return pltpu.make_async_copy(
            table_hbm.at[idx_v.at[c // 8, c % 8]],
            rows_v.at[slot], sems[slot])

    for b in range(NBUF):
        gather_copy(b, b).start()

    def body(i, carry):
        for b in range(NBUF):
            c = i * NBUF + b
            gather_copy(c, b).wait()
            pltpu.sync_copy(
                rows_v.at[b, :, pl.ds(0, E)],
                out_hbm.at[pl.ds(c * B + wid * CHUNK, CHUNK)])
            nxt = c + NBUF

            @pl.when(nxt < TCH)
            def _():
                gather_copy(nxt, b).start()
        return carry

    lax.fori_loop(0, TCH // NBUF, body, 0)


@functools.cache
def _make_sc_gather(k):
    return pl.kernel(
        functools.partial(_gather_body, k * TT),
        out_type=jax.ShapeDtypeStruct((CROWS, E), jnp.float32),
        mesh=plsc.VectorSubcoreMesh(core_axis_name="c", subcore_axis_name="s"),
        scratch_types=[
            pltpu.VMEM((TT, 8, CHUNK), jnp.int32),
            pltpu.VMEM((NBUF, CHUNK, 2 * E), jnp.float32),
        ] + [pltpu.SemaphoreType.DMA] * NBUF,
        compiler_params=pltpu.CompilerParams(
            use_tc_tiling_on_sc=False, needs_layout_passes=False),
    )


def _lstm_body(emb_ref, wx_ref, wh_ref, b_ref,
               hin_ref, cin_ref, hout_ref, cout_ref, h_scr, c_scr):
    tt = pl.program_id(0)

    @pl.when(tt == 0)
    def _():
        h_scr[...] = hin_ref[...]
        c_scr[...] = cin_ref[...]

    h = h_scr[...]
    c = c_scr[...]
    for p in range(TPB):
        xt = emb_ref[p]

        def gate(k, h=h, xt=xt):
            return (
                lax.dot_general(xt, wx_ref[k], (((1,), (0,)), ((), ())),
                                preferred_element_type=jnp.float32)
                + lax.dot_general(h, wh_ref[k], (((1,), (0,)), ((), ())),
                                  preferred_element_type=jnp.float32)
                + b_ref[k]
            )

        i = jax.nn.sigmoid(gate(0))
        f = jax.nn.sigmoid(gate(1))
        g = jnp.tanh(gate(2))
        o = jax.nn.sigmoid(gate(3))
        c = f * c + i * g
        h = o * jnp.tanh(c)
    h_scr[...] = h
    c_scr[...] = c

    @pl.when(tt == TCH // TPB - 1)
    def _():
        hout_ref[...] = h
        cout_ref[...] = c


def _lstm_chunk(emb_c, wx2, wh2, bias2, h, c, interpret=False):
    state = pl.BlockSpec((B2, H2), lambda t: (0, 0))
    return pl.pallas_call(
        _lstm_body,
        grid=(TCH // TPB,),
        in_specs=[
            pl.BlockSpec((TPB, B2, E2), lambda t: (t, 0, 0)),
            pl.BlockSpec((4, E2, H2), lambda t: (0, 0, 0)),
            pl.BlockSpec((4, H2, H2), lambda t: (0, 0, 0)),
            pl.BlockSpec((4, 1, H2), lambda t: (0, 0, 0)),
            state, state,
        ],
        out_specs=(state, state),
        out_shape=(jax.ShapeDtypeStruct((B2, H2), jnp.float32),
                   jax.ShapeDtypeStruct((B2, H2), jnp.float32)),
        scratch_shapes=[
            pltpu.VMEM((B2, H2), jnp.float32),
            pltpu.VMEM((B2, H2), jnp.float32),
        ],
        compiler_params=pltpu.CompilerParams(
            dimension_semantics=("arbitrary",)),
        interpret=interpret,
    )(emb_c, wx2, wh2, bias2, h, c)


def _blockdiag(w):
    k4, K, Hh = w.shape
    z = jnp.zeros((k4, K, Hh), w.dtype)
    top = jnp.concatenate([w, z], axis=2)
    bot = jnp.concatenate([z, w], axis=2)
    return jnp.concatenate([top, bot], axis=1)


def kernel(x, emb, W_ih, W_hh, b_ih, b_hh):
    x4 = jnp.transpose(
        x.astype(jnp.int32).reshape(NW, CHUNK, T // 8, 8), (2, 0, 3, 1))
    wx = jnp.transpose(W_ih.reshape(4, H, E), (0, 2, 1))
    wh = jnp.transpose(W_hh.reshape(4, H, H), (0, 2, 1))
    bias = (b_ih + b_hh).reshape(4, 1, H)
    wx2, wh2 = _blockdiag(wx), _blockdiag(wh)
    bias2 = jnp.concatenate([bias, bias], axis=2)

    emb128 = jnp.concatenate([emb, jnp.zeros((V, E), jnp.float32)], axis=1)

    h = jnp.zeros((B2, H2), jnp.float32)
    c = jnp.zeros((B2, H2), jnp.float32)
    for k in range(NCHT):
        emb_c = _make_sc_gather(k)(emb128, x4).reshape(TCH, B2, E2)
        h, c = _lstm_chunk(emb_c, wx2, wh2, bias2, h, c)
    return h.reshape(B, H)
